# baseline (speedup 1.0000x reference)
import functools
import math

import jax
import jax.numpy as jnp
from jax import lax
from jax.experimental import pallas as pl
from jax.experimental.pallas import tpu as pltpu

N_ENT = 100000
RANK = 64
MAX_NB = 50
B = 1024
TV = 2048


def _dense_small_kernel(lhs_ref, rel_ref, nb_ref, Ww_ref, Wb_ref, W2w_ref,
                        W2b_ref, Wow_ref, Wob_ref, Uow_ref, Uob_ref,
                        ec_ref, h_ref):
    lhs = lhs_ref[...]
    rel = rel_ref[...]
    nb = nb_ref[...]
    Ww = Ww_ref[...]
    w = (jnp.dot(lhs, Ww[:, :RANK].T, preferred_element_type=jnp.float32)
         + jnp.dot(rel, Ww[:, RANK:].T, preferred_element_type=jnp.float32)
         + Wb_ref[...])
    logits = jnp.sum(w[:, None, :] * nb, axis=2)
    m = jnp.max(logits, axis=1, keepdims=True)
    ex = jnp.exp(logits - m)
    alpha = ex / jnp.sum(ex, axis=1, keepdims=True)
    ec_pre = jnp.sum(alpha[:, :, None] * nb, axis=1)
    e_c = (jnp.dot(ec_pre, W2w_ref[...].T, preferred_element_type=jnp.float32)
           + W2b_ref[...])
    u = jnp.sum((lhs * rel) * Uow_ref[...], axis=1, keepdims=True) + Uob_ref[0, 0]
    wo = jnp.sum(e_c * Wow_ref[...], axis=1, keepdims=True) + Wob_ref[0, 0]
    g = 1.0 / (1.0 + jnp.exp(-(u + wo)))
    gated = g * e_c + (1.0 - g)
    ec_ref[...] = e_c
    h_ref[...] = lhs * rel * gated


def _vocab_kernel(h_ref, rhs_ref, out_ref):
    out_ref[...] = lax.dot_general(
        h_ref[...], rhs_ref[...], (((1,), (1,)), ((), ())),
        preferred_element_type=jnp.float32)


def kernel(x, slice_start, slice_end, tails, lhs_w, rel_w, rhs_w,
           W_w, W_b, W2_w, W2_b, Wo_w, Wo_b, Uo_w, Uo_b):
    subj = x[:, 0]
    lhs = lhs_w[subj]
    rel = rel_w[x[:, 1]]
    rhs_e = rhs_w[x[:, 2]]
    start = slice_start[subj]
    length = slice_end[subj] - start
    offs = jnp.arange(MAX_NB)
    pos = jnp.clip(start[:, None] + offs[None, :], 0, tails.shape[0] - 1)
    mask = offs[None, :] < length[:, None]
    idx = jnp.where(mask, tails[pos], 0)
    nb_E = rhs_w[idx]

    BBLK = 64
    ec, h = pl.pallas_call(
        _dense_small_kernel,
        grid=(B // BBLK,),
        in_specs=[
            pl.BlockSpec((BBLK, RANK), lambda i: (i, 0)),
            pl.BlockSpec((BBLK, RANK), lambda i: (i, 0)),
            pl.BlockSpec((BBLK, MAX_NB, RANK), lambda i: (i, 0, 0)),
            pl.BlockSpec((RANK, 2 * RANK), lambda i: (0, 0)),
            pl.BlockSpec((1, RANK), lambda i: (0, 0)),
            pl.BlockSpec((RANK, RANK), lambda i: (0, 0)),
            pl.BlockSpec((1, RANK), lambda i: (0, 0)),
            pl.BlockSpec((1, RANK), lambda i: (0, 0)),
            pl.BlockSpec((1, 1), lambda i: (0, 0)),
            pl.BlockSpec((1, RANK), lambda i: (0, 0)),
            pl.BlockSpec((1, 1), lambda i: (0, 0)),
        ],
        out_specs=(
            pl.BlockSpec((BBLK, RANK), lambda i: (i, 0)),
            pl.BlockSpec((BBLK, RANK), lambda i: (i, 0)),
        ),
        out_shape=(
            jax.ShapeDtypeStruct((B, RANK), jnp.float32),
            jax.ShapeDtypeStruct((B, RANK), jnp.float32),
        ),
    )(lhs, rel, nb_E, W_w, W_b.reshape(1, RANK), W2_w,
      W2_b.reshape(1, RANK), Wo_w.reshape(1, RANK), Wo_b.reshape(1, 1),
      Uo_w.reshape(1, RANK), Uo_b.reshape(1, 1))

    grid = (N_ENT + TV - 1) // TV
    tot = pl.pallas_call(
        _vocab_kernel,
        grid=(grid,),
        in_specs=[
            pl.BlockSpec((B, RANK), lambda i: (0, 0)),
            pl.BlockSpec((TV, RANK), lambda i: (i, 0)),
        ],
        out_specs=pl.BlockSpec((B, TV), lambda i: (0, i)),
        out_shape=jax.ShapeDtypeStruct((B, N_ENT), jnp.float32),
    )(h, rhs_w)

    return (tot, (lhs, rel, rhs_e, ec))
